# baseline (speedup 1.0000x reference)
import jax
import jax.numpy as jnp
from jax.experimental import pallas as pl
from jax.experimental.pallas import tpu as pltpu

BATCH = 16384
FIELDS = 4
EMBED = 64
BLK = 2048


def _fm_body(first_ref, second_ref, bias_ref, out_ref):
    x = second_ref[...]
    f = first_ref[...]
    s = (jax.lax.slice(x, (0, 0), (BLK, 64))
         + jax.lax.slice(x, (0, 64), (BLK, 128))
         + jax.lax.slice(x, (0, 128), (BLK, 192))
         + jax.lax.slice(x, (0, 192), (BLK, 256)))
    sq = jnp.sum(x * x, axis=1, keepdims=True)
    s2 = jnp.sum(s * s, axis=1, keepdims=True)
    ft = jnp.sum(f, axis=1, keepdims=True)
    out_ref[...] = bias_ref[0, 0] + ft + 0.5 * (s2 - sq)


def kernel(first_embeddings, second_embeddings, bias):
    x = second_embeddings.reshape(BATCH, FIELDS * EMBED)
    b2 = bias.reshape(1, 1)
    out = pl.pallas_call(
        _fm_body,
        grid=(BATCH // BLK,),
        in_specs=[
            pl.BlockSpec((BLK, FIELDS), lambda i: (i, 0)),
            pl.BlockSpec((BLK, FIELDS * EMBED), lambda i: (i, 0)),
            pl.BlockSpec((1, 1), lambda i: (0, 0)),
        ],
        out_specs=pl.BlockSpec((BLK, 1), lambda i: (i, 0)),
        out_shape=jax.ShapeDtypeStruct((BATCH, 1), jnp.float32),
    )(first_embeddings, x, b2)
    return out.reshape(BATCH)
